# baseline (speedup 1.0000x reference)
import jax
import jax.numpy as jnp
from jax.experimental import pallas as pl

_N = 10
_H = 512
_E = 160
_K = 3


def _fused(src_ref, dst_ref, x_ref, w_all_ref, b_all_ref,
           sa_w1_ref, sa_w2_ref, sa_w3_ref, bs_ref, vs_wT_ref,
           fc_w_ref, fc_b_ref, gcn_w_ref, gcn_b_ref,
           bp_ref, bm_ref):
    f32 = jnp.float32

    def conv_mm(X, i):
        w = w_all_ref[i * _H:(i + 1) * _H, :]
        out = jax.lax.dot_general(X, w, (((1,), (1,)), ((), ())),
                                  preferred_element_type=f32)
        return out + b_all_ref[i:i + 1, :]

    X = x_ref[...]
    for lvl in range(4):
        h = jnp.maximum(conv_mm(X, 2 * lvl), 0.0)
        h = jnp.maximum(conv_mm(h, 2 * lvl + 1), 0.0)
        X = jnp.maximum(h + X, 0.0)
    emb = X

    left = jnp.sum(emb * sa_w2_ref[...], axis=1, keepdims=True) * sa_w1_ref[0, 0]
    right = jnp.sum(emb * sa_w3_ref[...], axis=1, keepdims=True)
    sa_x = jax.lax.dot_general(left, right, (((1,), (1,)), ((), ())),
                               preferred_element_type=f32)
    sa_x = sa_x + bs_ref[...]
    sa_s = jnp.dot(jax.nn.sigmoid(sa_x), vs_wT_ref[...],
                   preferred_element_type=f32)

    n_iota = jax.lax.broadcasted_iota(jnp.int32, (_N, _E), 0)
    src_oh = (n_iota == src_ref[...]).astype(f32)
    dst_oh = (n_iota == dst_ref[...]).astype(f32)
    deg_out = jnp.sum(src_oh, axis=1, keepdims=True)
    deg_in = jnp.sum(dst_oh, axis=1, keepdims=True)
    norm_out = jnp.where(deg_out > 0,
                         jax.lax.rsqrt(jnp.maximum(deg_out, 1e-12)), 0.0)
    norm_in = jnp.where(deg_in > 0,
                        jax.lax.rsqrt(jnp.maximum(deg_in, 1e-12)), 0.0)
    A = jax.lax.dot_general(dst_oh, src_oh, (((1,), (1,)), ((), ())),
                            preferred_element_type=f32)
    h = emb * norm_out
    agg = jnp.dot(A, h, preferred_element_type=f32) * norm_in
    g_emb = jnp.dot(agg, gcn_w_ref[...], preferred_element_type=f32) + gcn_b_ref[...]

    sa_ag = jnp.dot(sa_s, g_emb, preferred_element_type=f32)
    logits = jnp.sum(sa_ag * fc_w_ref[...], axis=1, keepdims=True) + fc_b_ref[0, 0]
    scores = jax.nn.sigmoid(logits)

    e1 = jnp.exp(scores)
    bp_ref[...] = e1 / jnp.sum(e1)
    e2 = jnp.exp(1.0 - scores)
    bm_ref[...] = e2 / jnp.sum(e2)


@jax.jit
def kernel(x, edge_index, tcn_params, sa_w1, sa_w2, sa_w3, bs, vs_w,
           fc_w, fc_b, gcn_w, gcn_b):
    w_all = jnp.concatenate(
        [w[:, :, _K - 1] for (w1, b1, w2, b2) in tcn_params for w in (w1, w2)],
        axis=0)
    b_all = jnp.stack(
        [b for (w1, b1, w2, b2) in tcn_params for b in (b1, b2)], axis=0)
    ins = [edge_index[0:1, :], edge_index[1:2, :], x[:, :, 0], w_all, b_all,
           sa_w1, sa_w2.T, sa_w3, bs[None, :], vs_w.T,
           fc_w, fc_b[None, :], gcn_w, gcn_b[None, :]]

    bp, bm = pl.pallas_call(
        _fused,
        out_shape=[jax.ShapeDtypeStruct((_N, 1), jnp.float32),
                   jax.ShapeDtypeStruct((_N, 1), jnp.float32)],
    )(*ins)
    return bp[:, 0], bm[:, 0]


# speedup vs baseline: 1.1952x; 1.1952x over previous
import jax
import jax.numpy as jnp
from jax.experimental import pallas as pl
from jax.experimental.pallas import tpu as pltpu

_N = 10
_H = 512
_E = 160
_K = 3


def _fused(src_ref, dst_ref, x_ref,
           b10, b20, b11, b21, b12, b22, b13, b23,
           sa_w1_ref, sa_w2_ref, sa_w3_ref, bs_ref, vs_wT_ref,
           fc_w_ref, fc_b_ref, gcn_w_ref, gcn_b_ref,
           w10, w20, w11, w21, w12, w22, w13, w23,
           bp_ref, bm_ref):
    f32 = jnp.float32
    w_refs = (w10, w20, w11, w21, w12, w22, w13, w23)
    b_refs = (b10, b20, b11, b21, b12, b22, b13, b23)

    X = x_ref[...]
    for i in range(8):
        out = jax.lax.dot_general(X, w_refs[i][...], (((1,), (1,)), ((), ())),
                                  preferred_element_type=f32)
        out = out + b_refs[i][...]
        if i % 2 == 0:
            X_res, X = X, jnp.maximum(out, 0.0)
        else:
            X = jnp.maximum(jnp.maximum(out, 0.0) + X_res, 0.0)
    emb = X

    left = jnp.sum(emb * sa_w2_ref[...], axis=1, keepdims=True) * sa_w1_ref[0, 0]
    right = jnp.sum(emb * sa_w3_ref[...], axis=1, keepdims=True)
    sa_x = jax.lax.dot_general(left, right, (((1,), (1,)), ((), ())),
                               preferred_element_type=f32)
    sa_x = sa_x + bs_ref[...]
    sa_s = jnp.dot(jax.nn.sigmoid(sa_x), vs_wT_ref[...],
                   preferred_element_type=f32)

    n_iota = jax.lax.broadcasted_iota(jnp.int32, (_N, _E), 0)
    src_oh = (n_iota == src_ref[...]).astype(f32)
    dst_oh = (n_iota == dst_ref[...]).astype(f32)
    deg_out = jnp.sum(src_oh, axis=1, keepdims=True)
    deg_in = jnp.sum(dst_oh, axis=1, keepdims=True)
    norm_out = jnp.where(deg_out > 0,
                         jax.lax.rsqrt(jnp.maximum(deg_out, 1e-12)), 0.0)
    norm_in = jnp.where(deg_in > 0,
                        jax.lax.rsqrt(jnp.maximum(deg_in, 1e-12)), 0.0)
    A = jax.lax.dot_general(dst_oh, src_oh, (((1,), (1,)), ((), ())),
                            preferred_element_type=f32)
    h = emb * norm_out
    agg = jnp.dot(A, h, preferred_element_type=f32) * norm_in
    g_emb = jnp.dot(agg, gcn_w_ref[...], preferred_element_type=f32) + gcn_b_ref[...]

    sa_ag = jnp.dot(sa_s, g_emb, preferred_element_type=f32)
    logits = jnp.sum(sa_ag * fc_w_ref[...], axis=1, keepdims=True) + fc_b_ref[0, 0]
    scores = jax.nn.sigmoid(logits)

    e1 = jnp.exp(scores)
    bp_ref[...] = e1 / jnp.sum(e1)
    e2 = jnp.exp(1.0 - scores)
    bm_ref[...] = e2 / jnp.sum(e2)


@jax.jit
def kernel(x, edge_index, tcn_params, sa_w1, sa_w2, sa_w3, bs, vs_w,
           fc_w, fc_b, gcn_w, gcn_b):
    vmem_ins = [edge_index[0:1, :], edge_index[1:2, :], x[:, :, 0]]
    vmem_ins += [b[None, :] for (w1, b1, w2, b2) in tcn_params for b in (b1, b2)]
    vmem_ins += [sa_w1, sa_w2.T, sa_w3, bs[None, :], vs_w.T,
                 fc_w, fc_b[None, :], gcn_w, gcn_b[None, :]]
    w_ins = [w[:, :, _K - 1] for (w1, b1, w2, b2) in tcn_params
             for w in (w1, w2)]

    bp, bm = pl.pallas_call(
        _fused,
        out_shape=[jax.ShapeDtypeStruct((_N, 1), jnp.float32),
                   jax.ShapeDtypeStruct((_N, 1), jnp.float32)],
    )(*vmem_ins, *w_ins)
    return bp[:, 0], bm[:, 0]
